# Initial kernel scaffold; baseline (speedup 1.0000x reference)
import jax
import jax.numpy as jnp
from jax.experimental import pallas as pl


def kernel(atom, mulliken_charges, distance_rbf, connectivity, emb, W_init, b_init, W_eu1, b_eu1, W_eu2, b_eu2, W_me1, b_me1, W_me2, b_me2, W_af, b_af, W_st1, b_st1, W_st2, b_st2):
    raise NotImplementedError("write your pallas kernel here")


# speedup vs baseline: 2.1960x; 2.1960x over previous
import functools

import jax
import jax.numpy as jnp
from jax import lax
from jax.experimental import pallas as pl
from jax.experimental.pallas import tpu as pltpu
from jax.experimental.pallas import tpu_sc as plsc

_LOG2 = 0.6931471805599453


def _ssp(x):
    return jnp.maximum(x, 0.0) + jnp.log1p(jnp.exp(-jnp.abs(x))) - _LOG2


def _atom_embed_body(atom_ref, q_ref, emb_ref, out_ref):
    a = atom_ref[...]
    nz = emb_ref.shape[0]
    ids = lax.broadcasted_iota(jnp.int32, (a.shape[0], nz), 1)
    onehot = (a == ids).astype(jnp.float32)
    he = jnp.dot(onehot, emb_ref[...], preferred_element_type=jnp.float32)
    lane = lax.broadcasted_iota(jnp.int32, he.shape, 1)
    out_ref[...] = he + jnp.where(lane == he.shape[1] - 1, q_ref[...], 0.0)


def _build_x_atom(atom2d, charges, emb_pad, bn):
    n, _ = atom2d.shape
    nz, nb = emb_pad.shape
    grid = (n // bn,)
    return pl.pallas_call(
        _atom_embed_body,
        grid=grid,
        in_specs=[
            pl.BlockSpec((bn, 1), lambda i: (i, 0)),
            pl.BlockSpec((bn, 1), lambda i: (i, 0)),
            pl.BlockSpec((nz, nb), lambda i: (0, 0)),
        ],
        out_specs=pl.BlockSpec((bn, nb), lambda i: (i, 0)),
        out_shape=jax.ShapeDtypeStruct((n, nb), jnp.float32),
    )(atom2d, charges, emb_pad)


def _sc_gather_call(table, src, dst):
    n, d = table.shape
    e = src.shape[0]
    nw = 32
    per_w = e // nw
    ch = 80
    n_it = per_w // ch
    assert per_w * nw == e and n_it * ch == per_w

    mesh = plsc.VectorSubcoreMesh(core_axis_name="c", subcore_axis_name="s")

    @functools.partial(
        pl.kernel,
        mesh=mesh,
        out_type=[
            jax.ShapeDtypeStruct((e, d), jnp.float32),
            jax.ShapeDtypeStruct((e, d), jnp.float32),
        ],
        scratch_types=[
            pltpu.VMEM((ch,), jnp.int32),
            pltpu.VMEM((ch,), jnp.int32),
            pltpu.VMEM((ch, d), jnp.float32),
            pltpu.VMEM((ch, d), jnp.float32),
            pltpu.SemaphoreType.DMA,
            pltpu.SemaphoreType.DMA,
        ],
    )
    def gather_k(table_h, src_h, dst_h, out_s, out_d,
                 idx_s, idx_d, rows_s, rows_d, sem_s, sem_d):
        wid = lax.axis_index("s") * 2 + lax.axis_index("c")
        base = wid * per_w

        def body(j, carry):
            off = pl.multiple_of(base + j * ch, 8)
            pltpu.sync_copy(src_h.at[pl.ds(off, ch)], idx_s)
            pltpu.sync_copy(dst_h.at[pl.ds(off, ch)], idx_d)
            c1 = pltpu.async_copy(table_h.at[idx_s], rows_s, sem_s)
            c2 = pltpu.async_copy(table_h.at[idx_d], rows_d, sem_d)
            c1.wait()
            c2.wait()
            pltpu.sync_copy(rows_s, out_s.at[pl.ds(off, ch)])
            pltpu.sync_copy(rows_d, out_d.at[pl.ds(off, ch)])
            return carry

        lax.fori_loop(0, n_it, body, 0)

    return gather_k(table, src, dst)


def _sc_scatter_call(msgs, dsti, zeros, n_pad):
    e, d = msgs.shape
    per_sc = e // 2
    per_tile_e = per_sc // 16
    rows_t = n_pad // 16
    ch = 80
    n_it = per_tile_e // ch
    assert n_it * ch == per_tile_e and rows_t * 16 == n_pad and rows_t % 8 == 0

    mesh = plsc.VectorSubcoreMesh(core_axis_name="c", subcore_axis_name="s")

    @functools.partial(
        pl.kernel,
        mesh=mesh,
        out_type=jax.ShapeDtypeStruct((2 * n_pad, d), jnp.float32),
        scratch_types=[
            pltpu.VMEM((ch,), jnp.int32),
            pltpu.VMEM((ch, d), jnp.float32),
            pltpu.VMEM_SHARED((n_pad, d), jnp.float32),
        ],
    )
    def scatter_k(msg_h, dst_h, zeros_h, out_h, idx_v, rows_v, acc):
        c = lax.axis_index("c")
        s = lax.axis_index("s")
        pltpu.sync_copy(zeros_h, acc.at[pl.ds(s * rows_t, rows_t)])
        plsc.subcore_barrier()
        base = c * per_sc + s * per_tile_e

        def body(j, carry):
            off = pl.multiple_of(base + j * ch, 8)
            pltpu.sync_copy(dst_h.at[pl.ds(off, ch)], idx_v)
            pltpu.sync_copy(msg_h.at[pl.ds(off, ch)], rows_v)
            pltpu.sync_copy(rows_v, acc.at[idx_v], add=True)
            return carry

        lax.fori_loop(0, n_it, body, 0)
        plsc.subcore_barrier()
        pltpu.sync_copy(
            acc.at[pl.ds(s * rows_t, rows_t)],
            out_h.at[pl.ds(c * n_pad + s * rows_t, rows_t)],
        )

    return scatter_k(msgs, dsti, zeros)


def _edge_mlp_body(rbf_ref, xs_ref, xd_ref, wi_ref, bi_ref, w1a_ref, w1b_ref,
                   w1c_ref, b1_ref, w2_ref, b2_ref, wm1_ref, bm1_ref, wm2_ref,
                   bm2_ref, wa_ref, ba_ref, bond_ref, msg_ref):
    f32 = jnp.float32
    xb = _ssp(jnp.dot(rbf_ref[...], wi_ref[...], preferred_element_type=f32)
              + bi_ref[...])
    xs = xs_ref[...]
    xd = xd_ref[...]
    h = jnp.dot(xs, w1a_ref[...], preferred_element_type=f32)
    h = h + jnp.dot(xd, w1b_ref[...], preferred_element_type=f32)
    h = h + jnp.dot(xb, w1c_ref[...], preferred_element_type=f32)
    h = _ssp(h + b1_ref[...])
    xb2 = jnp.dot(h, w2_ref[...], preferred_element_type=f32) + b2_ref[...]
    bond_ref[...] = xb2
    m = _ssp(jnp.dot(xb2, wm1_ref[...], preferred_element_type=f32) + bm1_ref[...])
    m = _ssp(jnp.dot(m, wm2_ref[...], preferred_element_type=f32) + bm2_ref[...])
    sm = jnp.dot(xs, wa_ref[...], preferred_element_type=f32) + ba_ref[...]
    msg_ref[...] = m * sm


def _edge_mlp_call(rbf, xs, xd, wi, bi, w1a, w1b, w1c, b1, w2, b2, wm1, bm1,
                   wm2, bm2, wa, ba, be):
    e, k = rbf.shape
    nb = wi.shape[1]
    grid = (e // be,)

    def row(bs):
        return pl.BlockSpec(bs, lambda i: (i, 0))

    def full(a):
        return pl.BlockSpec(a.shape, lambda i: (0, 0))

    return pl.pallas_call(
        _edge_mlp_body,
        grid=grid,
        in_specs=[row((be, k)), row((be, nb)), row((be, nb)),
                  full(wi), full(bi), full(w1a), full(w1b), full(w1c),
                  full(b1), full(w2), full(b2), full(wm1), full(bm1),
                  full(wm2), full(bm2), full(wa), full(ba)],
        out_specs=[row((be, nb)), row((be, nb))],
        out_shape=[jax.ShapeDtypeStruct((e, nb), jnp.float32),
                   jax.ShapeDtypeStruct((e, nb), jnp.float32)],
    )(rbf, xs, xd, wi, bi, w1a, w1b, w1c, b1, w2, b2, wm1, bm1, wm2, bm2,
      wa, ba)


def _node_update_body(p0_ref, p1_ref, xa_ref, w1_ref, b1_ref, w2_ref, b2_ref,
                      out_ref):
    f32 = jnp.float32
    agg = p0_ref[...] + p1_ref[...]
    t = _ssp(jnp.dot(agg, w1_ref[...], preferred_element_type=f32) + b1_ref[...])
    out_ref[...] = (xa_ref[...] + jnp.dot(t, w2_ref[...], preferred_element_type=f32)
                    + b2_ref[...])


def _node_update_call(p0, p1, xa, w1, b1, w2, b2, bn):
    n, nb = xa.shape
    grid = (n // bn,)

    def row():
        return pl.BlockSpec((bn, nb), lambda i: (i, 0))

    def full(a):
        return pl.BlockSpec(a.shape, lambda i: (0, 0))

    return pl.pallas_call(
        _node_update_body,
        grid=grid,
        in_specs=[row(), row(), row(), full(w1), full(b1), full(w2), full(b2)],
        out_specs=row(),
        out_shape=jax.ShapeDtypeStruct((n, nb), jnp.float32),
    )(p0, p1, xa, w1, b1, w2, b2)


def kernel(atom, mulliken_charges, distance_rbf, connectivity, emb, W_init,
           b_init, W_eu1, b_eu1, W_eu2, b_eu2, W_me1, b_me1, W_me2, b_me2,
           W_af, b_af, W_st1, b_st1, W_st2, b_st2):
    n = atom.shape[0]
    e, _ = distance_rbf.shape
    nb = W_init.shape[1]

    src = connectivity[:, 0]
    dst = connectivity[:, 1]
    emb_pad = jnp.pad(emb, ((0, 0), (0, 1)))

    bn = 1000 if n % 1000 == 0 else n
    be = 2560 if e % 2560 == 0 else e

    x_atom = _build_x_atom(atom.reshape(n, 1), mulliken_charges, emb_pad, bn)

    xs, xd = _sc_gather_call(x_atom, src, dst)

    b2d = lambda b: b.reshape(1, -1)
    bond, msgs = _edge_mlp_call(
        distance_rbf, xs, xd, W_init, b2d(b_init),
        W_eu1[:nb], W_eu1[nb:2 * nb], W_eu1[2 * nb:], b2d(b_eu1),
        W_eu2, b2d(b_eu2), W_me1, b2d(b_me1), W_me2, b2d(b_me2),
        W_af, b2d(b_af), be)

    n_pad = ((n + 127) // 128) * 128
    zeros = jnp.zeros((n_pad // 16, nb), jnp.float32)
    parts = _sc_scatter_call(msgs, dst, zeros, n_pad)
    p0 = parts[:n]
    p1 = parts[n_pad:n_pad + n]

    x_out = _node_update_call(p0, p1, x_atom, W_st1, b2d(b_st1), W_st2,
                              b2d(b_st2), bn)
    return (x_out, bond)
